# Initial kernel scaffold; baseline (speedup 1.0000x reference)
import jax
import jax.numpy as jnp
from jax.experimental import pallas as pl


def kernel(value_feature, text_feature, edge_index, edge_weight, idx, fc1_w, fc1_b, fc2_w, fc2_b, relu_w, relu_b, conv0_w, conv0_b, conv1_w, conv1_b, fc3_w, fc3_b):
    raise NotImplementedError("write your pallas kernel here")


# speedup vs baseline: 8.5181x; 8.5181x over previous
import functools

import jax
import jax.numpy as jnp
from jax import lax
from jax.experimental import pallas as pl
from jax.experimental.pallas import tpu as pltpu
from jax.experimental.pallas import tpu_sc as plsc

N = 10000
NP = 10240
E = 320000
FEAT = 128
IDX_LEN = 1024
DW = 16

NC = 2
NS = 16
L = 16
NW = NC * NS
EPW = E // NW
CH = 80
NCH = EPW // CH

_MESH = plsc.VectorSubcoreMesh(core_axis_name="c", subcore_axis_name="s")


def _worker_id():
    return lax.axis_index("s") * NC + lax.axis_index("c")


@functools.partial(
    pl.kernel,
    out_type=jax.ShapeDtypeStruct((NC, NP, DW), jnp.float32),
    mesh=_MESH,
    scratch_types=[
        pltpu.VMEM((CH,), jnp.int32),
        pltpu.VMEM((CH,), jnp.float32),
        pltpu.VMEM((CH, DW), jnp.float32),
        pltpu.VMEM_SHARED((NP, DW), jnp.float32),
    ],
)
def _deg_kernel(dst_hbm, ew_hbm, out_hbm, dst_v, ew_v, rows_v, acc_sh):
    cid = lax.axis_index("c")
    sid = lax.axis_index("s")
    wid = _worker_id()
    rpw = NP // NS

    zero = jnp.zeros((L,), jnp.float32)
    for k in range(CH):
        rows_v[k, :] = zero
    for j in range(rpw // CH):
        pltpu.sync_copy(rows_v, acc_sh.at[pl.ds(sid * rpw + j * CH, CH)])
    plsc.subcore_barrier()

    def body(j, _):
        base = wid * EPW + j * CH
        pltpu.sync_copy(dst_hbm.at[pl.ds(base, CH)], dst_v)
        pltpu.sync_copy(ew_hbm.at[pl.ds(base, CH)], ew_v)
        for g in range(CH // L):
            wv = ew_v[pl.ds(g * L, L)]
            for l in range(L):
                rows_v[g * L + l, :] = jnp.full((L,), wv[l], jnp.float32)
        pltpu.sync_copy(rows_v, acc_sh.at[dst_v], add=True)
        return 0

    lax.fori_loop(0, NCH, body, 0)
    plsc.subcore_barrier()
    pltpu.sync_copy(acc_sh.at[pl.ds(sid * rpw, rpw)],
                    out_hbm.at[cid, pl.ds(sid * rpw, rpw)])


@functools.partial(
    pl.kernel,
    out_type=jax.ShapeDtypeStruct((NC, NP, FEAT), jnp.float32),
    mesh=_MESH,
    scratch_types=[
        pltpu.VMEM((CH,), jnp.int32),
        pltpu.VMEM((CH,), jnp.int32),
        pltpu.VMEM((CH,), jnp.float32),
        pltpu.VMEM((CH, FEAT), jnp.float32),
        pltpu.VMEM_SHARED((NP, FEAT), jnp.float32),
        pltpu.SemaphoreType.DMA,
    ],
)
def _agg_kernel(g_hbm, src_hbm, dst_hbm, ew_hbm, out_hbm,
                src_v, dst_v, ew_v, rows_v, acc_sh, sem):
    cid = lax.axis_index("c")
    sid = lax.axis_index("s")
    wid = _worker_id()
    rpw = NP // NS

    zero = jnp.zeros((L,), jnp.float32)
    for k in range(CH):
        for c in range(FEAT // L):
            rows_v[k, pl.ds(c * L, L)] = zero
    for j in range(rpw // CH):
        pltpu.sync_copy(rows_v, acc_sh.at[pl.ds(sid * rpw + j * CH, CH)])
    plsc.subcore_barrier()

    def body(j, _):
        base = wid * EPW + j * CH
        pltpu.sync_copy(src_hbm.at[pl.ds(base, CH)], src_v)
        pltpu.sync_copy(dst_hbm.at[pl.ds(base, CH)], dst_v)
        pltpu.sync_copy(ew_hbm.at[pl.ds(base, CH)], ew_v)
        pltpu.async_copy(g_hbm.at[src_v], rows_v, sem).wait()
        for g in range(CH // L):
            wv = ew_v[pl.ds(g * L, L)]
            for l in range(L):
                k = g * L + l
                w = wv[l]
                for c in range(FEAT // L):
                    sl = pl.ds(c * L, L)
                    rows_v[k, sl] = rows_v[k, sl] * w
        pltpu.sync_copy(rows_v, acc_sh.at[dst_v], add=True)
        return 0

    lax.fori_loop(0, NCH, body, 0)
    plsc.subcore_barrier()
    pltpu.sync_copy(acc_sh.at[pl.ds(sid * rpw, rpw)],
                    out_hbm.at[cid, pl.ds(sid * rpw, rpw)])


@functools.partial(
    pl.kernel,
    out_type=jax.ShapeDtypeStruct((IDX_LEN, FEAT), jnp.float32),
    mesh=_MESH,
    scratch_types=[
        pltpu.VMEM((IDX_LEN // NW,), jnp.int32),
        pltpu.VMEM((IDX_LEN // NW, FEAT), jnp.float32),
        pltpu.SemaphoreType.DMA,
    ],
)
def _gather_kernel(z_hbm, idx_hbm, out_hbm, idx_v, rows_v, sem):
    bpw = IDX_LEN // NW
    base = _worker_id() * bpw
    pltpu.sync_copy(idx_hbm.at[pl.ds(base, bpw)], idx_v)
    pltpu.async_copy(z_hbm.at[idx_v], rows_v, sem).wait()
    pltpu.sync_copy(rows_v, out_hbm.at[pl.ds(base, bpw)])


_BLK = 512
_GRID = NP // _BLK


def _dinv_of(degp_ref):
    deg = degp_ref[0, :, 0:1] + degp_ref[1, :, 0:1] + 1.0
    return lax.rsqrt(deg)


def _front_body(value_ref, text_ref, degp_ref, fc1wt, fc1b, fc2wt, fc2b,
                rw1t, rw2t, relub, w0t, g0_ref, h0_ref):
    v = jnp.dot(value_ref[...], fc1wt[...],
                preferred_element_type=jnp.float32) + fc1b[...]
    t = jnp.dot(text_ref[...], fc2wt[...],
                preferred_element_type=jnp.float32) + fc2b[...]
    pre = (jnp.dot(v, rw1t[...], preferred_element_type=jnp.float32)
           + jnp.dot(t, rw2t[...], preferred_element_type=jnp.float32)
           + relub[...])
    f = jnp.where(pre >= 0, pre, 0.01 * pre)
    h0 = jnp.dot(f, w0t[...], preferred_element_type=jnp.float32)
    h0_ref[...] = h0
    g0_ref[...] = _dinv_of(degp_ref) * h0


def _mid_body(parts_ref, h0_ref, degp_ref, w1t, b0, b1, g1_ref, s1_ref):
    dinv = _dinv_of(degp_ref)
    a1 = parts_ref[0] + parts_ref[1]
    feat1 = dinv * a1 + dinv * dinv * h0_ref[...] + b0[...]
    h1 = jnp.dot(feat1, w1t[...], preferred_element_type=jnp.float32)
    g1_ref[...] = dinv * h1
    s1_ref[...] = dinv * dinv * h1 + b1[...]


def _last_body(parts_ref, s1_ref, degp_ref, fc3tp, fc3bp, z_ref):
    dinv = _dinv_of(degp_ref)
    feat2 = dinv * (parts_ref[0] + parts_ref[1]) + s1_ref[...]
    z_ref[...] = jnp.dot(feat2, fc3tp[...],
                         preferred_element_type=jnp.float32) + fc3bp[...]


def _row_spec(width):
    return pl.BlockSpec((_BLK, width), lambda i: (i, 0))


def _parts_spec(width):
    return pl.BlockSpec((NC, _BLK, width), lambda i: (0, i, 0))


def _w_spec(r, c):
    return pl.BlockSpec((r, c), lambda i: (0, 0))


_front_call = pl.pallas_call(
    _front_body,
    grid=(_GRID,),
    in_specs=[
        _row_spec(16), _row_spec(768), _parts_spec(DW),
        _w_spec(16, FEAT), _w_spec(1, FEAT), _w_spec(768, FEAT),
        _w_spec(1, FEAT), _w_spec(FEAT, FEAT), _w_spec(FEAT, FEAT),
        _w_spec(1, FEAT), _w_spec(FEAT, FEAT),
    ],
    out_specs=[_row_spec(FEAT), _row_spec(FEAT)],
    out_shape=[jax.ShapeDtypeStruct((NP, FEAT), jnp.float32)] * 2,
)

_mid_call = pl.pallas_call(
    _mid_body,
    grid=(_GRID,),
    in_specs=[
        _parts_spec(FEAT), _row_spec(FEAT), _parts_spec(DW),
        _w_spec(FEAT, FEAT), _w_spec(1, FEAT), _w_spec(1, FEAT),
    ],
    out_specs=[_row_spec(FEAT), _row_spec(FEAT)],
    out_shape=[jax.ShapeDtypeStruct((NP, FEAT), jnp.float32)] * 2,
)

_last_call = pl.pallas_call(
    _last_body,
    grid=(_GRID,),
    in_specs=[
        _parts_spec(FEAT), _row_spec(FEAT), _parts_spec(DW),
        _w_spec(FEAT, FEAT), _w_spec(1, FEAT),
    ],
    out_specs=_row_spec(FEAT),
    out_shape=jax.ShapeDtypeStruct((NP, FEAT), jnp.float32),
)


def kernel(value_feature, text_feature, edge_index, edge_weight, idx,
           fc1_w, fc1_b, fc2_w, fc2_b, relu_w, relu_b,
           conv0_w, conv0_b, conv1_w, conv1_b, fc3_w, fc3_b):
    pad = NP - N
    value_p = jnp.pad(value_feature, ((0, pad), (0, 0)))
    text_p = jnp.pad(text_feature, ((0, pad), (0, 0)))
    src = edge_index[0]
    dst = edge_index[1]

    fc1wt = fc1_w.T
    fc2wt = fc2_w.T
    rw1t = relu_w[:, :FEAT].T
    rw2t = relu_w[:, FEAT:].T
    w0t = conv0_w.T
    w1t = conv1_w.T
    fc3tp = jnp.pad(fc3_w.T, ((0, 0), (0, FEAT - fc3_w.shape[0])))
    fc3bp = jnp.pad(fc3_b, (0, FEAT - fc3_b.shape[0])).reshape(1, FEAT)
    fc1b = fc1_b.reshape(1, FEAT)
    fc2b = fc2_b.reshape(1, FEAT)
    relub = relu_b.reshape(1, FEAT)
    b0 = conv0_b.reshape(1, FEAT)
    b1 = conv1_b.reshape(1, FEAT)

    degp = _deg_kernel(dst, edge_weight)
    g0, h0 = _front_call(value_p, text_p, degp, fc1wt, fc1b, fc2wt, fc2b,
                         rw1t, rw2t, relub, w0t)
    parts1 = _agg_kernel(g0, src, dst, edge_weight)
    g1, s1 = _mid_call(parts1, h0, degp, w1t, b0, b1)
    parts2 = _agg_kernel(g1, src, dst, edge_weight)
    z = _last_call(parts2, s1, degp, fc3tp, fc3bp)
    gathered = _gather_kernel(z, idx)
    return gathered[:, :fc3_w.shape[0]]


# speedup vs baseline: 9.0875x; 1.0668x over previous
import functools

import jax
import jax.numpy as jnp
from jax import lax
from jax.experimental import pallas as pl
from jax.experimental.pallas import tpu as pltpu
from jax.experimental.pallas import tpu_sc as plsc

N = 10000
NP = 10240
E = 320000
FEAT = 128
IDX_LEN = 1024
DW = 16

NC = 2
NS = 16
L = 16
NW = NC * NS
CH = 128
NCH = 80
EPW = NCH * CH
E2 = NW * EPW

_MESH = plsc.VectorSubcoreMesh(core_axis_name="c", subcore_axis_name="s")


def _worker_id():
    return lax.axis_index("s") * NC + lax.axis_index("c")


@functools.partial(
    pl.kernel,
    out_type=jax.ShapeDtypeStruct((NC, NP, DW), jnp.float32),
    mesh=_MESH,
    scratch_types=[
        pltpu.VMEM((NCH, CH), jnp.int32),
        pltpu.VMEM((NCH, CH), jnp.float32),
        pltpu.VMEM((CH, DW), jnp.float32),
        pltpu.VMEM_SHARED((NP, DW), jnp.float32),
    ],
)
def _deg_kernel(dst_hbm, ew_hbm, out_hbm, dst_all, ew_all, rows_v, acc_sh):
    cid = lax.axis_index("c")
    sid = lax.axis_index("s")
    wid = _worker_id()
    rpw = NP // NS

    pltpu.sync_copy(dst_hbm.at[wid], dst_all)
    pltpu.sync_copy(ew_hbm.at[wid], ew_all)
    zero = jnp.zeros((L,), jnp.float32)
    for k in range(CH):
        rows_v[k, :] = zero
    for j in range(rpw // CH):
        pltpu.sync_copy(rows_v, acc_sh.at[pl.ds(sid * rpw + j * CH, CH)])
    plsc.subcore_barrier()

    def body(j, _):
        for g in range(CH // L):
            wv = ew_all[j, pl.ds(g * L, L)]
            for l in range(L):
                rows_v[g * L + l, :] = jnp.full((L,), wv[l], jnp.float32)
        pltpu.sync_copy(rows_v, acc_sh.at[dst_all.at[j]], add=True)
        return 0

    lax.fori_loop(0, NCH, body, 0)
    plsc.subcore_barrier()
    pltpu.sync_copy(acc_sh.at[pl.ds(sid * rpw, rpw)],
                    out_hbm.at[cid, pl.ds(sid * rpw, rpw)])


@functools.partial(
    pl.kernel,
    out_type=jax.ShapeDtypeStruct((NC, NP, FEAT), jnp.float32),
    mesh=_MESH,
    scratch_types=[
        pltpu.VMEM((NCH, CH), jnp.int32),
        pltpu.VMEM((CH,), jnp.int32),
        pltpu.VMEM((CH,), jnp.int32),
        pltpu.VMEM((CH,), jnp.float32),
        pltpu.VMEM((CH,), jnp.float32),
        pltpu.VMEM((CH, FEAT), jnp.float32),
        pltpu.VMEM((CH, FEAT), jnp.float32),
        pltpu.VMEM_SHARED((NP, FEAT), jnp.float32),
        pltpu.SemaphoreType.DMA,
        pltpu.SemaphoreType.DMA,
    ],
)
def _agg_kernel(g_hbm, src_hbm, dst_hbm, ew_hbm, out_hbm,
                src_all, dst_a, dst_b, ew_a, ew_b, rows_a, rows_b,
                acc_sh, sem_a, sem_b):
    cid = lax.axis_index("c")
    sid = lax.axis_index("s")
    wid = _worker_id()
    rpw = NP // NS
    rows = (rows_a, rows_b)
    dsts = (dst_a, dst_b)
    ews = (ew_a, ew_b)
    sems = (sem_a, sem_b)

    pltpu.sync_copy(src_hbm.at[wid], src_all)

    def zero_body(r, _):
        z = jnp.zeros((L,), jnp.float32)
        for c in range(FEAT // L):
            rows_a[r, pl.ds(c * L, L)] = z
        return 0

    lax.fori_loop(0, CH, zero_body, 0)
    for j in range(rpw // CH):
        pltpu.sync_copy(rows_a, acc_sh.at[pl.ds(sid * rpw + j * CH, CH)])
    plsc.subcore_barrier()

    def scale(buf, ewb):
        def sbody(g, _):
            wv = ewb[pl.ds(g * L, L)]
            for l in range(L):
                w = wv[l]
                k = g * L + l
                for c in range(FEAT // L):
                    sl = pl.ds(c * L, L)
                    buf[k, sl] = buf[k, sl] * w
            return 0
        lax.fori_loop(0, CH // L, sbody, 0)

    pltpu.async_copy(g_hbm.at[src_all.at[0]], rows_a, sem_a)

    def body(i, _):
        for b in range(2):
            j = 2 * i + b
            jn = lax.rem(j + 1, NCH)
            pltpu.async_copy(g_hbm.at[src_all.at[jn]], rows[1 - b], sems[1 - b])
            pltpu.sync_copy(dst_hbm.at[wid, j], dsts[b])
            pltpu.sync_copy(ew_hbm.at[wid, j], ews[b])
            pltpu.make_async_copy(g_hbm.at[src_all.at[0]], rows[b], sems[b]).wait()
            scale(rows[b], ews[b])
            pltpu.sync_copy(rows[b], acc_sh.at[dsts[b]], add=True)
        return 0

    lax.fori_loop(0, NCH // 2, body, 0)
    pltpu.make_async_copy(g_hbm.at[src_all.at[0]], rows_a, sem_a).wait()

    plsc.subcore_barrier()
    pltpu.sync_copy(acc_sh.at[pl.ds(sid * rpw, rpw)],
                    out_hbm.at[cid, pl.ds(sid * rpw, rpw)])


@functools.partial(
    pl.kernel,
    out_type=jax.ShapeDtypeStruct((IDX_LEN, FEAT), jnp.float32),
    mesh=_MESH,
    scratch_types=[
        pltpu.VMEM((IDX_LEN // NW,), jnp.int32),
        pltpu.VMEM((IDX_LEN // NW, FEAT), jnp.float32),
        pltpu.SemaphoreType.DMA,
    ],
)
def _gather_kernel(z_hbm, idx_hbm, out_hbm, idx_v, rows_v, sem):
    bpw = IDX_LEN // NW
    base = _worker_id() * bpw
    pltpu.sync_copy(idx_hbm.at[pl.ds(base, bpw)], idx_v)
    pltpu.async_copy(z_hbm.at[idx_v], rows_v, sem).wait()
    pltpu.sync_copy(rows_v, out_hbm.at[pl.ds(base, bpw)])


_BLK = 512
_GRID = NP // _BLK


def _dinv_of(degp_ref):
    deg = degp_ref[0, :, 0:1] + degp_ref[1, :, 0:1] + 1.0
    return lax.rsqrt(deg)


def _front_body(value_ref, text_ref, degp_ref, fc1wt, fc1b, fc2wt, fc2b,
                rw1t, rw2t, relub, w0t, g0_ref, h0_ref):
    v = jnp.dot(value_ref[...], fc1wt[...],
                preferred_element_type=jnp.float32) + fc1b[...]
    t = jnp.dot(text_ref[...], fc2wt[...],
                preferred_element_type=jnp.float32) + fc2b[...]
    pre = (jnp.dot(v, rw1t[...], preferred_element_type=jnp.float32)
           + jnp.dot(t, rw2t[...], preferred_element_type=jnp.float32)
           + relub[...])
    f = jnp.where(pre >= 0, pre, 0.01 * pre)
    h0 = jnp.dot(f, w0t[...], preferred_element_type=jnp.float32)
    h0_ref[...] = h0
    g0_ref[...] = _dinv_of(degp_ref) * h0


def _mid_body(parts_ref, h0_ref, degp_ref, w1t, b0, b1, g1_ref, s1_ref):
    dinv = _dinv_of(degp_ref)
    a1 = parts_ref[0] + parts_ref[1]
    feat1 = dinv * a1 + dinv * dinv * h0_ref[...] + b0[...]
    h1 = jnp.dot(feat1, w1t[...], preferred_element_type=jnp.float32)
    g1_ref[...] = dinv * h1
    s1_ref[...] = dinv * dinv * h1 + b1[...]


def _last_body(parts_ref, s1_ref, degp_ref, fc3tp, fc3bp, z_ref):
    dinv = _dinv_of(degp_ref)
    feat2 = dinv * (parts_ref[0] + parts_ref[1]) + s1_ref[...]
    z_ref[...] = jnp.dot(feat2, fc3tp[...],
                         preferred_element_type=jnp.float32) + fc3bp[...]


def _row_spec(width):
    return pl.BlockSpec((_BLK, width), lambda i: (i, 0))


def _parts_spec(width):
    return pl.BlockSpec((NC, _BLK, width), lambda i: (0, i, 0))


def _w_spec(r, c):
    return pl.BlockSpec((r, c), lambda i: (0, 0))


_front_call = pl.pallas_call(
    _front_body,
    grid=(_GRID,),
    in_specs=[
        _row_spec(16), _row_spec(768), _parts_spec(DW),
        _w_spec(16, FEAT), _w_spec(1, FEAT), _w_spec(768, FEAT),
        _w_spec(1, FEAT), _w_spec(FEAT, FEAT), _w_spec(FEAT, FEAT),
        _w_spec(1, FEAT), _w_spec(FEAT, FEAT),
    ],
    out_specs=[_row_spec(FEAT), _row_spec(FEAT)],
    out_shape=[jax.ShapeDtypeStruct((NP, FEAT), jnp.float32)] * 2,
)

_mid_call = pl.pallas_call(
    _mid_body,
    grid=(_GRID,),
    in_specs=[
        _parts_spec(FEAT), _row_spec(FEAT), _parts_spec(DW),
        _w_spec(FEAT, FEAT), _w_spec(1, FEAT), _w_spec(1, FEAT),
    ],
    out_specs=[_row_spec(FEAT), _row_spec(FEAT)],
    out_shape=[jax.ShapeDtypeStruct((NP, FEAT), jnp.float32)] * 2,
)

_last_call = pl.pallas_call(
    _last_body,
    grid=(_GRID,),
    in_specs=[
        _parts_spec(FEAT), _row_spec(FEAT), _parts_spec(DW),
        _w_spec(FEAT, FEAT), _w_spec(1, FEAT),
    ],
    out_specs=_row_spec(FEAT),
    out_shape=jax.ShapeDtypeStruct((NP, FEAT), jnp.float32),
)


def kernel(value_feature, text_feature, edge_index, edge_weight, idx,
           fc1_w, fc1_b, fc2_w, fc2_b, relu_w, relu_b,
           conv0_w, conv0_b, conv1_w, conv1_b, fc3_w, fc3_b):
    pad = NP - N
    value_p = jnp.pad(value_feature, ((0, pad), (0, 0)))
    text_p = jnp.pad(text_feature, ((0, pad), (0, 0)))
    epad = E2 - E
    src = jnp.pad(edge_index[0], (0, epad)).reshape(NW, NCH, CH)
    dst = jnp.pad(edge_index[1], (0, epad)).reshape(NW, NCH, CH)
    eww = jnp.pad(edge_weight, (0, epad)).reshape(NW, NCH, CH)

    fc1wt = fc1_w.T
    fc2wt = fc2_w.T
    rw1t = relu_w[:, :FEAT].T
    rw2t = relu_w[:, FEAT:].T
    w0t = conv0_w.T
    w1t = conv1_w.T
    fc3tp = jnp.pad(fc3_w.T, ((0, 0), (0, FEAT - fc3_w.shape[0])))
    fc3bp = jnp.pad(fc3_b, (0, FEAT - fc3_b.shape[0])).reshape(1, FEAT)
    fc1b = fc1_b.reshape(1, FEAT)
    fc2b = fc2_b.reshape(1, FEAT)
    relub = relu_b.reshape(1, FEAT)
    b0 = conv0_b.reshape(1, FEAT)
    b1 = conv1_b.reshape(1, FEAT)

    degp = _deg_kernel(dst, eww)
    g0, h0 = _front_call(value_p, text_p, degp, fc1wt, fc1b, fc2wt, fc2b,
                         rw1t, rw2t, relub, w0t)
    parts1 = _agg_kernel(g0, src, dst, eww)
    g1, s1 = _mid_call(parts1, h0, degp, w1t, b0, b1)
    parts2 = _agg_kernel(g1, src, dst, eww)
    z = _last_call(parts2, s1, degp, fc3tp, fc3bp)
    gathered = _gather_kernel(z, idx)
    return gathered[:, :fc3_w.shape[0]]


# speedup vs baseline: 10.0328x; 1.1040x over previous
import functools

import jax
import jax.numpy as jnp
from jax import lax
from jax.experimental import pallas as pl
from jax.experimental.pallas import tpu as pltpu
from jax.experimental.pallas import tpu_sc as plsc

N = 10000
NP = 10240
E = 320000
FEAT = 128
IDX_LEN = 1024
DW = 16

NC = 2
NS = 16
L = 16
NW = NC * NS
CH = 128
NCH0 = 52
NCH1 = 106
NCHMX = 106
E2 = NS * (NCH0 + NCH1) * CH

_MESH = plsc.VectorSubcoreMesh(core_axis_name="c", subcore_axis_name="s")


def _worker_id():
    return lax.axis_index("s") * NC + lax.axis_index("c")


@functools.partial(
    pl.kernel,
    out_type=jax.ShapeDtypeStruct((NC, NP, DW), jnp.float32),
    mesh=_MESH,
    scratch_types=[
        pltpu.VMEM((NCHMX, CH), jnp.int32),
        pltpu.VMEM((NCHMX, CH), jnp.float32),
        pltpu.VMEM((CH, DW), jnp.float32),
        pltpu.VMEM_SHARED((NP, DW), jnp.float32),
    ],
)
def _deg_kernel(dst_hbm, ew_hbm, out_hbm, dst_all, ew_all, rows_v, acc_sh):
    cid = lax.axis_index("c")
    sid = lax.axis_index("s")
    wid = _worker_id()
    rpw = NP // NS

    pltpu.sync_copy(dst_hbm.at[wid], dst_all)
    pltpu.sync_copy(ew_hbm.at[wid], ew_all)
    zero = jnp.zeros((L,), jnp.float32)
    for k in range(CH):
        rows_v[k, :] = zero
    for j in range(rpw // CH):
        pltpu.sync_copy(rows_v, acc_sh.at[pl.ds(sid * rpw + j * CH, CH)])
    plsc.subcore_barrier()

    def body(j, _):
        for g in range(CH // L):
            wv = ew_all[j, pl.ds(g * L, L)]
            for l in range(L):
                rows_v[g * L + l, :] = jnp.full((L,), wv[l], jnp.float32)
        pltpu.sync_copy(rows_v, acc_sh.at[dst_all.at[j]], add=True)
        return 0

    @pl.when(cid == 0)
    def _():
        lax.fori_loop(0, NCH0, body, 0)

    @pl.when(cid != 0)
    def _():
        lax.fori_loop(0, NCH1, body, 0)

    plsc.subcore_barrier()
    pltpu.sync_copy(acc_sh.at[pl.ds(sid * rpw, rpw)],
                    out_hbm.at[cid, pl.ds(sid * rpw, rpw)])


@functools.partial(
    pl.kernel,
    out_type=jax.ShapeDtypeStruct((NC, NP, FEAT), jnp.float32),
    mesh=_MESH,
    scratch_types=[
        pltpu.VMEM((NCHMX, CH), jnp.int32),
        pltpu.VMEM((CH,), jnp.int32),
        pltpu.VMEM((CH,), jnp.int32),
        pltpu.VMEM((CH,), jnp.float32),
        pltpu.VMEM((CH,), jnp.float32),
        pltpu.VMEM((CH, FEAT), jnp.float32),
        pltpu.VMEM((CH, FEAT), jnp.float32),
        pltpu.VMEM_SHARED((NP, FEAT), jnp.float32),
        pltpu.SemaphoreType.DMA,
        pltpu.SemaphoreType.DMA,
    ],
)
def _agg_kernel(g_hbm, src_hbm, dst_hbm, ew_hbm, out_hbm,
                src_all, dst_a, dst_b, ew_a, ew_b, rows_a, rows_b,
                acc_sh, sem_a, sem_b):
    cid = lax.axis_index("c")
    sid = lax.axis_index("s")
    wid = _worker_id()
    rpw = NP // NS
    rows = (rows_a, rows_b)
    dsts = (dst_a, dst_b)
    ews = (ew_a, ew_b)
    sems = (sem_a, sem_b)

    pltpu.sync_copy(src_hbm.at[wid], src_all)

    def zero_body(r, _):
        z = jnp.zeros((L,), jnp.float32)
        for c in range(FEAT // L):
            rows_a[r, pl.ds(c * L, L)] = z
        return 0

    lax.fori_loop(0, CH, zero_body, 0)
    for j in range(rpw // CH):
        pltpu.sync_copy(rows_a, acc_sh.at[pl.ds(sid * rpw + j * CH, CH)])
    plsc.subcore_barrier()

    def scale(buf, ewb):
        def sbody(g, _):
            wv = ewb[pl.ds(g * L, L)]
            for l in range(L):
                w = wv[l]
                k = g * L + l
                for c in range(FEAT // L):
                    sl = pl.ds(c * L, L)
                    buf[k, sl] = buf[k, sl] * w
            return 0
        lax.fori_loop(0, CH // L, sbody, 0)

    def run(nch):
        pltpu.async_copy(g_hbm.at[src_all.at[0]], rows_a, sem_a)

        def body(i, _):
            for b in range(2):
                j = 2 * i + b
                jn = lax.rem(j + 1, nch)
                pltpu.async_copy(g_hbm.at[src_all.at[jn]], rows[1 - b],
                                 sems[1 - b])
                pltpu.sync_copy(dst_hbm.at[wid, j], dsts[b])
                pltpu.sync_copy(ew_hbm.at[wid, j], ews[b])
                pltpu.make_async_copy(g_hbm.at[src_all.at[0]], rows[b],
                                      sems[b]).wait()
                scale(rows[b], ews[b])
                pltpu.sync_copy(rows[b], acc_sh.at[dsts[b]], add=True)
            return 0

        lax.fori_loop(0, nch // 2, body, 0)
        pltpu.make_async_copy(g_hbm.at[src_all.at[0]], rows_a, sem_a).wait()

    @pl.when(cid == 0)
    def _():
        run(NCH0)

    @pl.when(cid != 0)
    def _():
        run(NCH1)

    plsc.subcore_barrier()
    pltpu.sync_copy(acc_sh.at[pl.ds(sid * rpw, rpw)],
                    out_hbm.at[cid, pl.ds(sid * rpw, rpw)])


@functools.partial(
    pl.kernel,
    out_type=jax.ShapeDtypeStruct((IDX_LEN, FEAT), jnp.float32),
    mesh=_MESH,
    scratch_types=[
        pltpu.VMEM((IDX_LEN // NW,), jnp.int32),
        pltpu.VMEM((IDX_LEN // NW, FEAT), jnp.float32),
        pltpu.SemaphoreType.DMA,
    ],
)
def _gather_kernel(z_hbm, idx_hbm, out_hbm, idx_v, rows_v, sem):
    bpw = IDX_LEN // NW
    base = _worker_id() * bpw
    pltpu.sync_copy(idx_hbm.at[pl.ds(base, bpw)], idx_v)
    pltpu.async_copy(z_hbm.at[idx_v], rows_v, sem).wait()
    pltpu.sync_copy(rows_v, out_hbm.at[pl.ds(base, bpw)])


_BLK = 512
_GRID = NP // _BLK


def _dinv_of(degp_ref):
    deg = degp_ref[0, :, 0:1] + degp_ref[1, :, 0:1] + 1.0
    return lax.rsqrt(deg)


def _front_body(value_ref, text_ref, degp_ref, fc1wt, fc1b, fc2wt, fc2b,
                rw1t, rw2t, relub, w0t, g0_ref, h0_ref):
    v = jnp.dot(value_ref[...], fc1wt[...],
                preferred_element_type=jnp.float32) + fc1b[...]
    t = jnp.dot(text_ref[...], fc2wt[...],
                preferred_element_type=jnp.float32) + fc2b[...]
    pre = (jnp.dot(v, rw1t[...], preferred_element_type=jnp.float32)
           + jnp.dot(t, rw2t[...], preferred_element_type=jnp.float32)
           + relub[...])
    f = jnp.where(pre >= 0, pre, 0.01 * pre)
    h0 = jnp.dot(f, w0t[...], preferred_element_type=jnp.float32)
    h0_ref[...] = h0
    g0_ref[...] = _dinv_of(degp_ref) * h0


def _mid_body(parts_ref, h0_ref, degp_ref, w1t, b0, b1, g1_ref, s1_ref):
    dinv = _dinv_of(degp_ref)
    a1 = parts_ref[0] + parts_ref[1]
    feat1 = dinv * a1 + dinv * dinv * h0_ref[...] + b0[...]
    h1 = jnp.dot(feat1, w1t[...], preferred_element_type=jnp.float32)
    g1_ref[...] = dinv * h1
    s1_ref[...] = dinv * dinv * h1 + b1[...]


def _last_body(parts_ref, s1_ref, degp_ref, fc3tp, fc3bp, z_ref):
    dinv = _dinv_of(degp_ref)
    feat2 = dinv * (parts_ref[0] + parts_ref[1]) + s1_ref[...]
    z_ref[...] = jnp.dot(feat2, fc3tp[...],
                         preferred_element_type=jnp.float32) + fc3bp[...]


def _row_spec(width):
    return pl.BlockSpec((_BLK, width), lambda i: (i, 0))


def _parts_spec(width):
    return pl.BlockSpec((NC, _BLK, width), lambda i: (0, i, 0))


def _w_spec(r, c):
    return pl.BlockSpec((r, c), lambda i: (0, 0))


_front_call = pl.pallas_call(
    _front_body,
    grid=(_GRID,),
    in_specs=[
        _row_spec(16), _row_spec(768), _parts_spec(DW),
        _w_spec(16, FEAT), _w_spec(1, FEAT), _w_spec(768, FEAT),
        _w_spec(1, FEAT), _w_spec(FEAT, FEAT), _w_spec(FEAT, FEAT),
        _w_spec(1, FEAT), _w_spec(FEAT, FEAT),
    ],
    out_specs=[_row_spec(FEAT), _row_spec(FEAT)],
    out_shape=[jax.ShapeDtypeStruct((NP, FEAT), jnp.float32)] * 2,
)

_mid_call = pl.pallas_call(
    _mid_body,
    grid=(_GRID,),
    in_specs=[
        _parts_spec(FEAT), _row_spec(FEAT), _parts_spec(DW),
        _w_spec(FEAT, FEAT), _w_spec(1, FEAT), _w_spec(1, FEAT),
    ],
    out_specs=[_row_spec(FEAT), _row_spec(FEAT)],
    out_shape=[jax.ShapeDtypeStruct((NP, FEAT), jnp.float32)] * 2,
)

_last_call = pl.pallas_call(
    _last_body,
    grid=(_GRID,),
    in_specs=[
        _parts_spec(FEAT), _row_spec(FEAT), _parts_spec(DW),
        _w_spec(FEAT, FEAT), _w_spec(1, FEAT),
    ],
    out_specs=_row_spec(FEAT),
    out_shape=jax.ShapeDtypeStruct((NP, FEAT), jnp.float32),
)


def kernel(value_feature, text_feature, edge_index, edge_weight, idx,
           fc1_w, fc1_b, fc2_w, fc2_b, relu_w, relu_b,
           conv0_w, conv0_b, conv1_w, conv1_b, fc3_w, fc3_b):
    pad = NP - N
    value_p = jnp.pad(value_feature, ((0, pad), (0, 0)))
    text_p = jnp.pad(text_feature, ((0, pad), (0, 0)))
    epad = E2 - E

    def _shard(flat):
        flat = jnp.pad(flat, (0, epad))
        segs = []
        off = 0
        for w in range(NW):
            n = NCH0 if w % NC == 0 else NCH1
            seg = flat[off:off + n * CH].reshape(n, CH)
            segs.append(jnp.pad(seg, ((0, NCHMX - n), (0, 0))))
            off += n * CH
        return jnp.stack(segs)

    src = _shard(edge_index[0])
    dst = _shard(edge_index[1])
    eww = _shard(edge_weight)

    fc1wt = fc1_w.T
    fc2wt = fc2_w.T
    rw1t = relu_w[:, :FEAT].T
    rw2t = relu_w[:, FEAT:].T
    w0t = conv0_w.T
    w1t = conv1_w.T
    fc3tp = jnp.pad(fc3_w.T, ((0, 0), (0, FEAT - fc3_w.shape[0])))
    fc3bp = jnp.pad(fc3_b, (0, FEAT - fc3_b.shape[0])).reshape(1, FEAT)
    fc1b = fc1_b.reshape(1, FEAT)
    fc2b = fc2_b.reshape(1, FEAT)
    relub = relu_b.reshape(1, FEAT)
    b0 = conv0_b.reshape(1, FEAT)
    b1 = conv1_b.reshape(1, FEAT)

    degp = _deg_kernel(dst, eww)
    g0, h0 = _front_call(value_p, text_p, degp, fc1wt, fc1b, fc2wt, fc2b,
                         rw1t, rw2t, relub, w0t)
    parts1 = _agg_kernel(g0, src, dst, eww)
    g1, s1 = _mid_call(parts1, h0, degp, w1t, b0, b1)
    parts2 = _agg_kernel(g1, src, dst, eww)
    z = _last_call(parts2, s1, degp, fc3tp, fc3bp)
    gathered = _gather_kernel(z, idx)
    return gathered[:, :fc3_w.shape[0]]


# speedup vs baseline: 13.2643x; 1.3221x over previous
import functools

import jax
import jax.numpy as jnp
from jax import lax
from jax.experimental import pallas as pl
from jax.experimental.pallas import tpu as pltpu
from jax.experimental.pallas import tpu_sc as plsc

N = 10000
NP = 10240
E = 320000
FEAT = 128
IDX_LEN = 1024
DW = 16

NC = 2
NS = 16
L = 16
NW = NC * NS
CH = 128
NCH0 = 106
NCH1 = 52
NCHMX = 106
E2 = NS * (NCH0 + NCH1) * CH

_MESH = plsc.VectorSubcoreMesh(core_axis_name="c", subcore_axis_name="s")


def _worker_id():
    return lax.axis_index("s") * NC + lax.axis_index("c")


@functools.partial(
    pl.kernel,
    out_type=jax.ShapeDtypeStruct((NC, NP, DW), jnp.float32),
    mesh=_MESH,
    scratch_types=[
        pltpu.VMEM((NCHMX, CH), jnp.int32),
        pltpu.VMEM((NCHMX, CH), jnp.float32),
        pltpu.VMEM((CH, DW), jnp.float32),
        pltpu.VMEM_SHARED((NP, DW), jnp.float32),
    ],
)
def _deg_kernel(dst_hbm, ew_hbm, out_hbm, dst_all, ew_all, rows_v, acc_sh):
    cid = lax.axis_index("c")
    sid = lax.axis_index("s")
    wid = _worker_id()
    rpw = NP // NS

    pltpu.sync_copy(dst_hbm.at[wid], dst_all)
    pltpu.sync_copy(ew_hbm.at[wid], ew_all)
    zero = jnp.zeros((L,), jnp.float32)
    for k in range(CH):
        rows_v[k, :] = zero
    for j in range(rpw // CH):
        pltpu.sync_copy(rows_v, acc_sh.at[pl.ds(sid * rpw + j * CH, CH)])
    plsc.subcore_barrier()

    def body(j, _):
        for g in range(CH // L):
            wv = ew_all[j, pl.ds(g * L, L)]
            for l in range(L):
                rows_v[g * L + l, :] = jnp.full((L,), wv[l], jnp.float32)
        pltpu.sync_copy(rows_v, acc_sh.at[dst_all.at[j]], add=True)
        return 0

    @pl.when(cid == 0)
    def _():
        lax.fori_loop(0, NCH0, body, 0)

    @pl.when(cid != 0)
    def _():
        lax.fori_loop(0, NCH1, body, 0)

    plsc.subcore_barrier()
    pltpu.sync_copy(acc_sh.at[pl.ds(sid * rpw, rpw)],
                    out_hbm.at[cid, pl.ds(sid * rpw, rpw)])


@functools.partial(
    pl.kernel,
    out_type=jax.ShapeDtypeStruct((NC, NP, FEAT), jnp.float32),
    mesh=_MESH,
    scratch_types=[
        pltpu.VMEM((NCHMX, CH), jnp.int32),
        pltpu.VMEM((CH,), jnp.int32),
        pltpu.VMEM((CH,), jnp.int32),
        pltpu.VMEM((CH,), jnp.float32),
        pltpu.VMEM((CH,), jnp.float32),
        pltpu.VMEM((CH, FEAT), jnp.float32),
        pltpu.VMEM((CH, FEAT), jnp.float32),
        pltpu.VMEM_SHARED((NP, FEAT), jnp.float32),
        pltpu.SemaphoreType.DMA,
        pltpu.SemaphoreType.DMA,
    ],
)
def _agg_kernel(g_hbm, src_hbm, dst_hbm, ew_hbm, out_hbm,
                src_all, dst_a, dst_b, ew_a, ew_b, rows_a, rows_b,
                acc_sh, sem_a, sem_b):
    cid = lax.axis_index("c")
    sid = lax.axis_index("s")
    wid = _worker_id()
    rpw = NP // NS
    rows = (rows_a, rows_b)
    dsts = (dst_a, dst_b)
    ews = (ew_a, ew_b)
    sems = (sem_a, sem_b)

    pltpu.sync_copy(src_hbm.at[wid], src_all)

    def zero_body(r, _):
        z = jnp.zeros((L,), jnp.float32)
        for c in range(FEAT // L):
            rows_a[r, pl.ds(c * L, L)] = z
        return 0

    lax.fori_loop(0, CH, zero_body, 0)
    for j in range(rpw // CH):
        pltpu.sync_copy(rows_a, acc_sh.at[pl.ds(sid * rpw + j * CH, CH)])
    plsc.subcore_barrier()

    def scale(buf, ewb):
        def sbody(g, _):
            wv = ewb[pl.ds(g * L, L)]
            for l in range(L):
                w = wv[l]
                k = g * L + l
                for c in range(FEAT // L):
                    sl = pl.ds(c * L, L)
                    buf[k, sl] = buf[k, sl] * w
            return 0
        lax.fori_loop(0, CH // L, sbody, 0)

    def run(nch):
        pltpu.async_copy(g_hbm.at[src_all.at[0]], rows_a, sem_a)

        def body(i, _):
            for b in range(2):
                j = 2 * i + b
                jn = lax.rem(j + 1, nch)
                pltpu.async_copy(g_hbm.at[src_all.at[jn]], rows[1 - b],
                                 sems[1 - b])
                pltpu.sync_copy(dst_hbm.at[wid, j], dsts[b])
                pltpu.sync_copy(ew_hbm.at[wid, j], ews[b])
                pltpu.make_async_copy(g_hbm.at[src_all.at[0]], rows[b],
                                      sems[b]).wait()
                scale(rows[b], ews[b])
                pltpu.sync_copy(rows[b], acc_sh.at[dsts[b]], add=True)
            return 0

        lax.fori_loop(0, nch // 2, body, 0)
        pltpu.make_async_copy(g_hbm.at[src_all.at[0]], rows_a, sem_a).wait()

    @pl.when(cid == 0)
    def _():
        run(NCH0)

    @pl.when(cid != 0)
    def _():
        run(NCH1)

    plsc.subcore_barrier()
    pltpu.sync_copy(acc_sh.at[pl.ds(sid * rpw, rpw)],
                    out_hbm.at[cid, pl.ds(sid * rpw, rpw)])


@functools.partial(
    pl.kernel,
    out_type=jax.ShapeDtypeStruct((IDX_LEN, FEAT), jnp.float32),
    mesh=_MESH,
    scratch_types=[
        pltpu.VMEM((IDX_LEN // NW,), jnp.int32),
        pltpu.VMEM((IDX_LEN // NW, FEAT), jnp.float32),
        pltpu.SemaphoreType.DMA,
    ],
)
def _gather_kernel(z_hbm, idx_hbm, out_hbm, idx_v, rows_v, sem):
    bpw = IDX_LEN // NW
    base = _worker_id() * bpw
    pltpu.sync_copy(idx_hbm.at[pl.ds(base, bpw)], idx_v)
    pltpu.async_copy(z_hbm.at[idx_v], rows_v, sem).wait()
    pltpu.sync_copy(rows_v, out_hbm.at[pl.ds(base, bpw)])


_BLK = 512
_GRID = NP // _BLK


def _dinv_of(degp_ref):
    deg = degp_ref[0, :, 0:1] + degp_ref[1, :, 0:1] + 1.0
    return lax.rsqrt(deg)


def _front_body(value_ref, text_ref, degp_ref, fc1wt, fc1b, fc2wt, fc2b,
                rw1t, rw2t, relub, w0t, g0_ref, h0_ref):
    v = jnp.dot(value_ref[...], fc1wt[...],
                preferred_element_type=jnp.float32) + fc1b[...]
    t = jnp.dot(text_ref[...], fc2wt[...],
                preferred_element_type=jnp.float32) + fc2b[...]
    pre = (jnp.dot(v, rw1t[...], preferred_element_type=jnp.float32)
           + jnp.dot(t, rw2t[...], preferred_element_type=jnp.float32)
           + relub[...])
    f = jnp.where(pre >= 0, pre, 0.01 * pre)
    h0 = jnp.dot(f, w0t[...], preferred_element_type=jnp.float32)
    h0_ref[...] = h0
    g0_ref[...] = _dinv_of(degp_ref) * h0


def _mid_body(parts_ref, h0_ref, degp_ref, w1t, b0, b1, g1_ref, s1_ref):
    dinv = _dinv_of(degp_ref)
    a1 = parts_ref[0] + parts_ref[1]
    feat1 = dinv * a1 + dinv * dinv * h0_ref[...] + b0[...]
    h1 = jnp.dot(feat1, w1t[...], preferred_element_type=jnp.float32)
    g1_ref[...] = dinv * h1
    s1_ref[...] = dinv * dinv * h1 + b1[...]


def _last_body(parts_ref, s1_ref, degp_ref, fc3tp, fc3bp, z_ref):
    dinv = _dinv_of(degp_ref)
    feat2 = dinv * (parts_ref[0] + parts_ref[1]) + s1_ref[...]
    z_ref[...] = jnp.dot(feat2, fc3tp[...],
                         preferred_element_type=jnp.float32) + fc3bp[...]


def _row_spec(width):
    return pl.BlockSpec((_BLK, width), lambda i: (i, 0))


def _parts_spec(width):
    return pl.BlockSpec((NC, _BLK, width), lambda i: (0, i, 0))


def _w_spec(r, c):
    return pl.BlockSpec((r, c), lambda i: (0, 0))


_front_call = pl.pallas_call(
    _front_body,
    grid=(_GRID,),
    in_specs=[
        _row_spec(16), _row_spec(768), _parts_spec(DW),
        _w_spec(16, FEAT), _w_spec(1, FEAT), _w_spec(768, FEAT),
        _w_spec(1, FEAT), _w_spec(FEAT, FEAT), _w_spec(FEAT, FEAT),
        _w_spec(1, FEAT), _w_spec(FEAT, FEAT),
    ],
    out_specs=[_row_spec(FEAT), _row_spec(FEAT)],
    out_shape=[jax.ShapeDtypeStruct((NP, FEAT), jnp.float32)] * 2,
)

_mid_call = pl.pallas_call(
    _mid_body,
    grid=(_GRID,),
    in_specs=[
        _parts_spec(FEAT), _row_spec(FEAT), _parts_spec(DW),
        _w_spec(FEAT, FEAT), _w_spec(1, FEAT), _w_spec(1, FEAT),
    ],
    out_specs=[_row_spec(FEAT), _row_spec(FEAT)],
    out_shape=[jax.ShapeDtypeStruct((NP, FEAT), jnp.float32)] * 2,
)

_last_call = pl.pallas_call(
    _last_body,
    grid=(_GRID,),
    in_specs=[
        _parts_spec(FEAT), _row_spec(FEAT), _parts_spec(DW),
        _w_spec(FEAT, FEAT), _w_spec(1, FEAT),
    ],
    out_specs=_row_spec(FEAT),
    out_shape=jax.ShapeDtypeStruct((NP, FEAT), jnp.float32),
)


def kernel(value_feature, text_feature, edge_index, edge_weight, idx,
           fc1_w, fc1_b, fc2_w, fc2_b, relu_w, relu_b,
           conv0_w, conv0_b, conv1_w, conv1_b, fc3_w, fc3_b):
    pad = NP - N
    value_p = jnp.pad(value_feature, ((0, pad), (0, 0)))
    text_p = jnp.pad(text_feature, ((0, pad), (0, 0)))
    epad = E2 - E

    def _shard(flat):
        flat = jnp.pad(flat, (0, epad))
        segs = []
        off = 0
        for w in range(NW):
            n = NCH0 if w % NC == 0 else NCH1
            seg = flat[off:off + n * CH].reshape(n, CH)
            segs.append(jnp.pad(seg, ((0, NCHMX - n), (0, 0))))
            off += n * CH
        return jnp.stack(segs)

    src = _shard(edge_index[0])
    dst = _shard(edge_index[1])
    eww = _shard(edge_weight)

    fc1wt = fc1_w.T
    fc2wt = fc2_w.T
    rw1t = relu_w[:, :FEAT].T
    rw2t = relu_w[:, FEAT:].T
    w0t = conv0_w.T
    w1t = conv1_w.T
    fc3tp = jnp.pad(fc3_w.T, ((0, 0), (0, FEAT - fc3_w.shape[0])))
    fc3bp = jnp.pad(fc3_b, (0, FEAT - fc3_b.shape[0])).reshape(1, FEAT)
    fc1b = fc1_b.reshape(1, FEAT)
    fc2b = fc2_b.reshape(1, FEAT)
    relub = relu_b.reshape(1, FEAT)
    b0 = conv0_b.reshape(1, FEAT)
    b1 = conv1_b.reshape(1, FEAT)

    degp = _deg_kernel(dst, eww)
    g0, h0 = _front_call(value_p, text_p, degp, fc1wt, fc1b, fc2wt, fc2b,
                         rw1t, rw2t, relub, w0t)
    parts1 = _agg_kernel(g0, src, dst, eww)
    g1, s1 = _mid_call(parts1, h0, degp, w1t, b0, b1)
    parts2 = _agg_kernel(g1, src, dst, eww)
    z = _last_call(parts2, s1, degp, fc3tp, fc3bp)
    gathered = _gather_kernel(z, idx)
    return gathered[:, :fc3_w.shape[0]]
